# baseline (speedup 1.0000x reference)
import jax
import jax.numpy as jnp
from jax import lax
from jax.experimental import pallas as pl
from jax.experimental.pallas import tpu as pltpu
from jax.experimental.pallas import tpu_sc as plsc

NCN = 4096
MAPW = NCN * NCN
WPAD = MAPW
RPAD0 = MAPW + 16384
RPAD1 = MAPW + 32768
TOTW = MAPW + 49152

K = 200000
P = 100000
Q = 100000

SCH = 104
KT = SCH * 128
KP = KT * 16

QCH = 52
PT = QCH * 128
PP = PT * 16

QT = 3200
QP = QT * 32
FV = QT // 16

ZCH = 16384
HALFW = MAPW // 2
TSLICE = HALFW // 16

_mesh = plsc.VectorSubcoreMesh(
    core_axis_name="c", subcore_axis_name="s", num_cores=2, num_subcores=16)


def _sc_body(gt_t, gt_s, ovl, q_t, q_s, tx_h, ty_h, tz_h, sx_h, sy_h, sz_h,
             consts,
             map_hbm, couts, fouts,
             zbuf, tgt_b, src_b, ovl_b, fine_b, sidx, qidx, qdst, acc_b,
             ones2_b, consts_v, semz, sems, semg):
    c = lax.axis_index("c")
    s = lax.axis_index("s")
    w = c * 16 + s
    lanes = lax.iota(jnp.int32, 16)
    zeros16 = jnp.zeros((16,), jnp.float32)
    ones16 = jnp.ones((16,), jnp.float32)

    scope = jax.named_scope
    pltpu.sync_copy(consts, consts_v)
    def fill_o(i, _):
        ones2_b[pl.ds(i * 16, 16)] = ones16
        return 0
    lax.fori_loop(0, KT // 16, fill_o, 0)

    with scope("p0_fill"):
        def fill_z(i, _):
            zbuf[pl.ds(i * 16, 16)] = zeros16
            return 0
        lax.fori_loop(0, ZCH // 16, fill_z, 0)

    half_base = c * HALFW
    tile_base = half_base + s * TSLICE

    def fire_zero(k, _):
        pltpu.async_copy(zbuf, map_hbm.at[pl.ds(tile_base + k * ZCH, ZCH)],
                         semz)
        return 0
    lax.fori_loop(0, TSLICE // ZCH, fire_zero, 0)
    rpad_c = jnp.where(c == 0, RPAD0, RPAD1)
    pltpu.async_copy(zbuf.at[pl.ds(0, 1024)],
                     map_hbm.at[pl.ds(rpad_c + s * 1024, 1024)], semz)

    kbase = s * KT
    pltpu.sync_copy(gt_t.at[pl.ds(kbase, KT)], tgt_b)
    pltpu.sync_copy(gt_s.at[pl.ds(kbase, KT)], src_b)
    pltpu.sync_copy(ovl.at[pl.ds(kbase, KT)], ovl_b)

    def mk_sidx(i, _):
        off = i * 16
        t = tgt_b[pl.ds(off, 16)]
        sr = src_b[pl.ds(off, 16)]
        ov = ovl_b[pl.ds(off, 16)]
        lin = t * NCN + sr
        valid = (ov > 0.0) & ((t >> 11) == c)
        pad = WPAD + (((off + lanes) * 32 + w) & 16383)
        sidx[pl.ds(off, 16)] = jnp.where(valid, lin, pad)
        return 0
    with scope("p1_sidx"):
        lax.fori_loop(0, KT // 16, mk_sidx, 0)

    def wait_zero(k, _):
        pltpu.make_async_copy(
            zbuf, map_hbm.at[pl.ds(tile_base + k * ZCH, ZCH)], semz).wait()
        return 0
    with scope("p2_zdrain"):
        lax.fori_loop(0, TSLICE // ZCH, wait_zero, 0)
    pltpu.make_async_copy(zbuf.at[pl.ds(0, 1024)],
                          map_hbm.at[pl.ds(rpad_c + s * 1024, 1024)],
                          semz).wait()
    plsc.subcore_barrier()

    with scope("p3_scatter"):
        for rnd in range(16):
            @pl.when(s == rnd)
            def _():
                pltpu.sync_copy(ones2_b, map_hbm.at[sidx])
            plsc.subcore_barrier()

    pbase = s * PT
    pltpu.sync_copy(q_t.at[pl.ds(pbase, PT)], tgt_b.at[pl.ds(0, PT)])
    pltpu.sync_copy(q_s.at[pl.ds(pbase, PT)], src_b.at[pl.ds(0, PT)])

    def mk_qidx(i, _):
        off = i * 16
        t = tgt_b[pl.ds(off, 16)]
        sr = src_b[pl.ds(off, 16)]
        lin = t * NCN + sr
        valid = (t >> 11) == c
        pad = rpad_c + (((off + lanes) * 32 + s) & 16383)
        qidx[pl.ds(off, 16)] = jnp.where(valid, lin, pad)
        return 0
    with scope("p4_qidx"):
        lax.fori_loop(0, PT // 16, mk_qidx, 0)


    with scope("p5_gfire"):
        pltpu.async_copy(map_hbm.at[qidx], qdst, semg)

    qbase = w * QT
    fb = [ovl_b.at[pl.ds(i * QT, QT)] for i in range(3)] + \
         [fine_b.at[pl.ds(i * QT, QT)] for i in range(3)]
    for i, h in enumerate((tx_h, ty_h, tz_h, sx_h, sy_h, sz_h)):
        pltpu.sync_copy(h.at[pl.ds(qbase, QT)], fb[i])
    cv = [consts_v[pl.ds(j * 16, 16)] for j in range(13)]

    def fine(i, facc):
        off = i * 16
        tx = fb[0][pl.ds(off, 16)]
        ty = fb[1][pl.ds(off, 16)]
        tz = fb[2][pl.ds(off, 16)]
        sx = fb[3][pl.ds(off, 16)]
        sy = fb[4][pl.ds(off, 16)]
        sz = fb[5][pl.ds(off, 16)]
        dx = cv[0] * sx + cv[1] * sy + cv[2] * sz + cv[9] - tx
        dy = cv[3] * sx + cv[4] * sy + cv[5] * sz + cv[10] - ty
        dz = cv[6] * sx + cv[7] * sy + cv[8] * sz + cv[11] - tz
        d2 = dx * dx + dy * dy + dz * dz
        return facc + jnp.where(d2 < cv[12], ones16, zeros16)
    with scope("p6_fine"):
        facc = lax.fori_loop(0, FV, fine, zeros16)

    with scope("p7_gdrain"):
        pltpu.make_async_copy(map_hbm.at[qidx], qdst, semg).wait()

    def csum(i, cacc):
        return cacc + qdst[pl.ds(i * 16, 16)]
    with scope("p8_csum"):
        cacc = lax.fori_loop(0, PT // 16, csum, zeros16)

    def clr_acc(i, _):
        acc_b[pl.ds(i * 16, 16)] = zeros16
        return 0
    lax.fori_loop(0, 16, clr_acc, 0)
    acc_b[pl.ds(0, 16)] = cacc
    acc_b[pl.ds(128, 16)] = facc
    pltpu.sync_copy(acc_b.at[pl.ds(0, 128)], couts.at[w])
    pltpu.sync_copy(acc_b.at[pl.ds(128, 128)], fouts.at[w])


@jax.jit
def _run(gt_t, gt_s, ovl, q_t, q_s, tx, ty, tz, sx, sy, sz, consts):
    f = pl.kernel(
        _sc_body,
        out_type=(
            jax.ShapeDtypeStruct((TOTW,), jnp.float32),
            jax.ShapeDtypeStruct((32, 128), jnp.float32),
            jax.ShapeDtypeStruct((32, 128), jnp.float32),
        ),
        mesh=_mesh,
        scratch_types=(
            pltpu.VMEM((ZCH,), jnp.float32),
            pltpu.VMEM((KT,), jnp.int32),
            pltpu.VMEM((KT,), jnp.int32),
            pltpu.VMEM((KT,), jnp.float32),
            pltpu.VMEM((3 * QT,), jnp.float32),
            pltpu.VMEM((KT,), jnp.int32),
            pltpu.VMEM((PT,), jnp.int32),
            pltpu.VMEM((PT,), jnp.float32),
            pltpu.VMEM((256,), jnp.float32),
            pltpu.VMEM((KT,), jnp.float32),
            pltpu.VMEM((208,), jnp.float32),
            pltpu.SemaphoreType.DMA,
            pltpu.SemaphoreType.DMA,
            pltpu.SemaphoreType.DMA,
        ),
    )
    return f(gt_t, gt_s, ovl, q_t, q_s, tx, ty, tz, sx, sy, sz, consts)


def kernel(tgt_nodes, src_nodes, src_node_feats, gt_node_corr_overlaps,
           gt_node_corr_indices, tgt_node_corr_indices, src_node_corr_indices,
           tgt_corr_points, src_corr_points, rot, trans):
    gti = gt_node_corr_indices.astype(jnp.int32)
    gt_t = jnp.concatenate([gti[:, 0], jnp.zeros((KP - K,), jnp.int32)])
    gt_s = jnp.concatenate([gti[:, 1], jnp.zeros((KP - K,), jnp.int32)])
    ovl = jnp.concatenate([gt_node_corr_overlaps,
                           jnp.zeros((KP - K,), jnp.float32)])
    q_t = jnp.concatenate([tgt_node_corr_indices.astype(jnp.int32),
                           jnp.full((PP - P,), NCN, jnp.int32)])
    q_s = jnp.concatenate([src_node_corr_indices.astype(jnp.int32),
                           jnp.zeros((PP - P,), jnp.int32)])
    tpts = jnp.concatenate([tgt_corr_points,
                            jnp.full((QP - Q, 3), 1e9, jnp.float32)]).T
    spts = jnp.concatenate([src_corr_points,
                            jnp.zeros((QP - Q, 3), jnp.float32)]).T
    consts = (jnp.concatenate([
        rot[0].reshape(9), trans[0].reshape(3),
        jnp.array([0.01], jnp.float32), jnp.zeros((3,), jnp.float32),
    ])[:13].reshape(13, 1) * jnp.ones((1, 16), jnp.float32)).reshape(208)

    _, couts, fouts = _run(gt_t, gt_s, ovl, q_t, q_s,
                           tpts[0], tpts[1], tpts[2],
                           spts[0], spts[1], spts[2], consts)

    c_precision = jnp.sum(couts) / jnp.float32(P)
    f_precision = jnp.sum(fouts) / jnp.float32(Q)
    fmr = f_precision > 0.05
    num_matches = jnp.array(Q, dtype=jnp.int32)
    return (c_precision, f_precision, fmr, num_matches)


# speedup vs baseline: 3.6960x; 3.6960x over previous
import jax
import jax.numpy as jnp
from jax import lax
from jax.experimental import pallas as pl
from jax.experimental.pallas import tpu as pltpu
from jax.experimental.pallas import tpu_sc as plsc

NCN = 4096
HALFP = NCN * NCN // 2
BIG = 0x40000000

K = 200000
P = 100000
Q = 100000

KT = 13312
KP = KT * 16
PT = 6656
PP = PT * 16
QT = 3200
QP = QT * 32

CH = 2048
KCH = [2048] * 6 + [1024]
PCH = [2048] * 3 + [512]
FCH = [1024] * 3 + [128]

SLW = 1294336
NSL = 7
WSP = SLW
RSP = SLW + 8192
TW = SLW + 16384
ZW = 4096
TZ = TW // 16

_mesh = plsc.VectorSubcoreMesh(
    core_axis_name="c", subcore_axis_name="s", num_cores=2, num_subcores=16)


def _sc_body(gt_t, gt_s, ovl, q_t, q_s, tx_h, ty_h, tz_h, sx_h, sy_h, sz_h,
             consts,
             couts, fouts,
             zbuf, sidx, qsidx, widx, qwidx, qdst, ones_b, fine_b, acc_b,
             consts_v, qmap_sh, semz, sems, semg):
    c = lax.axis_index("c")
    s = lax.axis_index("s")
    w = c * 16 + s
    lanes = lax.iota(jnp.int32, 16)
    zeros16 = jnp.zeros((16,), jnp.float32)
    ones16 = jnp.ones((16,), jnp.float32)
    scope = jax.named_scope

    pltpu.sync_copy(consts, consts_v)

    def fill_o(i, _):
        ones_b[pl.ds(i * 16, 16)] = ones16
        return 0
    lax.fori_loop(0, CH // 16, fill_o, 0)

    def fill_z(i, _):
        zbuf[pl.ds(i * 16, 16)] = zeros16
        return 0
    lax.fori_loop(0, ZW // 16, fill_z, 0)

    hbase = c * HALFP

    with scope("p1_sidx"):
        off0 = 0
        for n in KCH:
            pltpu.sync_copy(gt_t.at[pl.ds(s * KT + off0, n)],
                            widx.at[pl.ds(0, n)])
            pltpu.sync_copy(gt_s.at[pl.ds(s * KT + off0, n)],
                            qwidx.at[pl.ds(0, n)])
            pltpu.sync_copy(ovl.at[pl.ds(s * KT + off0, n)],
                            qdst.at[pl.ds(0, n)])

            def mk_s(i, _, off0=off0, __n=n):
                o = i * 16
                t = widx[pl.ds(o, 16)]
                sr = qwidx[pl.ds(o, 16)]
                ov = qdst[pl.ds(o, 16)]
                lin = t * NCN + sr - hbase
                valid = (ov > 0.0) & ((t >> 11) == c)
                sidx[pl.ds(off0 + o, 16)] = jnp.where(valid, lin, BIG)
                return 0
            lax.fori_loop(0, n // 16, mk_s, 0)
            off0 += n

    with scope("p2_qidx"):
        off0 = 0
        for n in PCH:
            pltpu.sync_copy(q_t.at[pl.ds(s * PT + off0, n)],
                            widx.at[pl.ds(0, n)])
            pltpu.sync_copy(q_s.at[pl.ds(s * PT + off0, n)],
                            qwidx.at[pl.ds(0, n)])

            def mk_q(i, _, off0=off0):
                o = i * 16
                t = widx[pl.ds(o, 16)]
                sr = qwidx[pl.ds(o, 16)]
                lin = t * NCN + sr - hbase
                qsidx[pl.ds(off0 + o, 16)] = jnp.where((t >> 11) == c,
                                                       lin, BIG)
                return 0
            lax.fori_loop(0, n // 16, mk_q, 0)
            off0 += n

    cacc = zeros16
    zbase = s * TZ
    for t_sl in range(NSL):
        base = t_sl * SLW

        with scope("p3_zero"):
            def fire_zero(k, _):
                pltpu.async_copy(zbuf,
                                 qmap_sh.at[pl.ds(zbase + k * ZW, ZW)], semz)
                return 0
            lax.fori_loop(0, TZ // ZW, fire_zero, 0)

            def wait_zero(k, _):
                pltpu.make_async_copy(
                    zbuf, qmap_sh.at[pl.ds(zbase + k * ZW, ZW)], semz).wait()
                return 0
            lax.fori_loop(0, TZ // ZW, wait_zero, 0)
        plsc.subcore_barrier()

        with scope("p4_scat"):
            fired = []
            off0 = 0
            for ch, n in enumerate(KCH):
                bk = (ch & 1) * CH
                if len(fired) >= 2:
                    fo, fn, fb_ = fired[len(fired) - 2]
                    pltpu.make_async_copy(
                        ones_b.at[pl.ds(0, fn)],
                        qmap_sh.at[widx.at[pl.ds(fb_, fn)]], sems).wait()

                def mk_w(i, _, off0=off0, bk=bk):
                    o = i * 16
                    d = sidx[pl.ds(off0 + o, 16)] - base
                    ins = (d >= 0) & (d < SLW)
                    pad = WSP + (((off0 + o + lanes) * 32 + w) & 8191)
                    widx[pl.ds(bk + o, 16)] = jnp.where(ins, d, pad)
                    return 0
                lax.fori_loop(0, n // 16, mk_w, 0)
                pltpu.async_copy(ones_b.at[pl.ds(0, n)],
                                 qmap_sh.at[widx.at[pl.ds(bk, n)]],
                                 sems, add=True)
                fired.append((off0, n, bk))
                off0 += n
            for fo, fn, fb_ in fired[len(fired) - 2:]:
                pltpu.make_async_copy(
                    ones_b.at[pl.ds(0, fn)],
                    qmap_sh.at[widx.at[pl.ds(fb_, fn)]], sems).wait()
        plsc.subcore_barrier()

        with scope("p5_gath"):
            live = []
            off0 = 0
            for ch, n in enumerate(PCH):
                bk = (ch & 1) * CH

                def mk_qw(i, _, off0=off0, bk=bk):
                    o = i * 16
                    d = qsidx[pl.ds(off0 + o, 16)] - base
                    ins = (d >= 0) & (d < SLW)
                    pad = RSP + (((off0 + o + lanes) * 32 + s) & 8191)
                    qwidx[pl.ds(bk + o, 16)] = jnp.where(ins, d, pad)
                    return 0
                lax.fori_loop(0, n // 16, mk_qw, 0)
                pltpu.async_copy(qmap_sh.at[qwidx.at[pl.ds(bk, n)]],
                                 qdst.at[pl.ds(bk, n)], semg)
                live.append((n, bk))
                if len(live) == 2:
                    fn, fb_ = live.pop(0)
                    pltpu.make_async_copy(
                        qmap_sh.at[qwidx.at[pl.ds(fb_, fn)]],
                        qdst.at[pl.ds(fb_, fn)], semg).wait()

                    def acc_f(i, a, fb_=fb_):
                        g = qdst[pl.ds(fb_ + i * 16, 16)]
                        return a + jnp.where(g > 0.0, 1.0, 0.0)
                    cacc = lax.fori_loop(0, fn // 16, acc_f, cacc)
                off0 += n
            for fn, fb_ in live:
                pltpu.make_async_copy(
                    qmap_sh.at[qwidx.at[pl.ds(fb_, fn)]],
                    qdst.at[pl.ds(fb_, fn)], semg).wait()

                def acc_f(i, a, fb_=fb_):
                    g = qdst[pl.ds(fb_ + i * 16, 16)]
                    return a + jnp.where(g > 0.0, 1.0, 0.0)
                cacc = lax.fori_loop(0, fn // 16, acc_f, cacc)
        plsc.subcore_barrier()

    cv = [consts_v[pl.ds(j * 16, 16)] for j in range(13)]
    facc = zeros16
    with scope("p6_fine"):
        off0 = 0
        for n in FCH:
            hs = (tx_h, ty_h, tz_h, sx_h, sy_h, sz_h)
            for i, h in enumerate(hs):
                pltpu.sync_copy(h.at[pl.ds(w * QT + off0, n)],
                                fine_b.at[pl.ds(i * 1024, n)])

            def fine(i, fa):
                o = i * 16
                tx = fine_b[pl.ds(o, 16)]
                ty = fine_b[pl.ds(1024 + o, 16)]
                tz = fine_b[pl.ds(2048 + o, 16)]
                sx = fine_b[pl.ds(3072 + o, 16)]
                sy = fine_b[pl.ds(4096 + o, 16)]
                sz = fine_b[pl.ds(5120 + o, 16)]
                dx = cv[0] * sx + cv[1] * sy + cv[2] * sz + cv[9] - tx
                dy = cv[3] * sx + cv[4] * sy + cv[5] * sz + cv[10] - ty
                dz = cv[6] * sx + cv[7] * sy + cv[8] * sz + cv[11] - tz
                d2 = dx * dx + dy * dy + dz * dz
                return fa + jnp.where(d2 < cv[12], ones16, zeros16)
            facc = lax.fori_loop(0, n // 16, fine, facc)
            off0 += n

    def clr_acc(i, _):
        acc_b[pl.ds(i * 16, 16)] = zeros16
        return 0
    lax.fori_loop(0, 16, clr_acc, 0)
    acc_b[pl.ds(0, 16)] = cacc
    acc_b[pl.ds(128, 16)] = facc
    pltpu.sync_copy(acc_b.at[pl.ds(0, 128)], couts.at[w])
    pltpu.sync_copy(acc_b.at[pl.ds(128, 128)], fouts.at[w])


@jax.jit
def _run(gt_t, gt_s, ovl, q_t, q_s, tx, ty, tz, sx, sy, sz, consts):
    f = pl.kernel(
        _sc_body,
        out_type=(
            jax.ShapeDtypeStruct((32, 128), jnp.float32),
            jax.ShapeDtypeStruct((32, 128), jnp.float32),
        ),
        mesh=_mesh,
        scratch_types=(
            pltpu.VMEM((ZW,), jnp.float32),
            pltpu.VMEM((KT,), jnp.int32),
            pltpu.VMEM((PT,), jnp.int32),
            pltpu.VMEM((2 * CH,), jnp.int32),
            pltpu.VMEM((2 * CH,), jnp.int32),
            pltpu.VMEM((2 * CH,), jnp.float32),
            pltpu.VMEM((CH,), jnp.float32),
            pltpu.VMEM((6 * 1024,), jnp.float32),
            pltpu.VMEM((256,), jnp.float32),
            pltpu.VMEM((208,), jnp.float32),
            pltpu.VMEM_SHARED((TW,), jnp.float32),
            pltpu.SemaphoreType.DMA,
            pltpu.SemaphoreType.DMA,
            pltpu.SemaphoreType.DMA,
        ),
    )
    return f(gt_t, gt_s, ovl, q_t, q_s, tx, ty, tz, sx, sy, sz, consts)


def kernel(tgt_nodes, src_nodes, src_node_feats, gt_node_corr_overlaps,
           gt_node_corr_indices, tgt_node_corr_indices, src_node_corr_indices,
           tgt_corr_points, src_corr_points, rot, trans):
    gti = gt_node_corr_indices.astype(jnp.int32)
    gt_t = jnp.concatenate([gti[:, 0], jnp.zeros((KP - K,), jnp.int32)])
    gt_s = jnp.concatenate([gti[:, 1], jnp.zeros((KP - K,), jnp.int32)])
    ovl = jnp.concatenate([gt_node_corr_overlaps,
                           jnp.zeros((KP - K,), jnp.float32)])
    q_t = jnp.concatenate([tgt_node_corr_indices.astype(jnp.int32),
                           jnp.full((PP - P,), NCN, jnp.int32)])
    q_s = jnp.concatenate([src_node_corr_indices.astype(jnp.int32),
                           jnp.zeros((PP - P,), jnp.int32)])
    tpts = jnp.concatenate([tgt_corr_points,
                            jnp.full((QP - Q, 3), 1e9, jnp.float32)]).T
    spts = jnp.concatenate([src_corr_points,
                            jnp.zeros((QP - Q, 3), jnp.float32)]).T
    consts = (jnp.concatenate([
        rot[0].reshape(9), trans[0].reshape(3),
        jnp.array([0.01], jnp.float32), jnp.zeros((3,), jnp.float32),
    ])[:13].reshape(13, 1) * jnp.ones((1, 16), jnp.float32)).reshape(208)

    couts, fouts = _run(gt_t, gt_s, ovl, q_t, q_s,
                        tpts[0], tpts[1], tpts[2],
                        spts[0], spts[1], spts[2], consts)

    c_precision = jnp.sum(couts) / jnp.float32(P)
    f_precision = jnp.sum(fouts) / jnp.float32(Q)
    fmr = f_precision > 0.05
    num_matches = jnp.array(Q, dtype=jnp.int32)
    return (c_precision, f_precision, fmr, num_matches)


# speedup vs baseline: 4.0039x; 1.0833x over previous
import jax
import jax.numpy as jnp
from jax import lax
from jax.experimental import pallas as pl
from jax.experimental.pallas import tpu as pltpu
from jax.experimental.pallas import tpu_sc as plsc

NCN = 4096
HALFP = NCN * NCN // 2
BIG = 0x40000000

K = 200000
P = 100000
Q = 100000

KT = 13312
KP = KT * 16
PT = 6656
PP = PT * 16
QT = 3200
QP = QT * 32

CH = 2048
KCH = [2048] * 6 + [1024]
PCH = [2048] * 3 + [512]
FCH = [1024] * 3 + [128]

SLW = 1425408
NSL = 6
WSP = SLW
RSP = SLW + 8192
TW = SLW + 16384
ZW = 4096
TZ = TW // 16

_mesh = plsc.VectorSubcoreMesh(
    core_axis_name="c", subcore_axis_name="s", num_cores=2, num_subcores=16)


def _sc_body(gt_t, gt_s, ovl, q_t, q_s, tx_h, ty_h, tz_h, sx_h, sy_h, sz_h,
             consts,
             couts, fouts,
             zbuf, sidx, qsidx, widx, qwidx, qdst, ones_b, acc_b,
             consts_v, qmap_sh, semz, sems, semg):
    c = lax.axis_index("c")
    s = lax.axis_index("s")
    w = c * 16 + s
    lanes = lax.iota(jnp.int32, 16)
    zeros16 = jnp.zeros((16,), jnp.float32)
    ones16 = jnp.ones((16,), jnp.float32)
    scope = jax.named_scope

    pltpu.sync_copy(consts, consts_v)

    def fill_o(i, _):
        ones_b[pl.ds(i * 16, 16)] = ones16
        return 0
    lax.fori_loop(0, CH // 16, fill_o, 0)

    def fill_z(i, _):
        zbuf[pl.ds(i * 16, 16)] = zeros16
        return 0
    lax.fori_loop(0, ZW // 16, fill_z, 0)

    hbase = c * HALFP

    with scope("p1_sidx"):
        off0 = 0
        for n in KCH:
            pltpu.sync_copy(gt_t.at[pl.ds(s * KT + off0, n)],
                            widx.at[pl.ds(0, n)])
            pltpu.sync_copy(gt_s.at[pl.ds(s * KT + off0, n)],
                            qwidx.at[pl.ds(0, n)])
            pltpu.sync_copy(ovl.at[pl.ds(s * KT + off0, n)],
                            qdst.at[pl.ds(0, n)])

            def mk_s(i, _, off0=off0, __n=n):
                o = i * 16
                t = widx[pl.ds(o, 16)]
                sr = qwidx[pl.ds(o, 16)]
                ov = qdst[pl.ds(o, 16)]
                lin = t * NCN + sr - hbase
                valid = (ov > 0.0) & ((t >> 11) == c)
                sidx[pl.ds(off0 + o, 16)] = jnp.where(valid, lin, BIG)
                return 0
            lax.fori_loop(0, n // 16, mk_s, 0)
            off0 += n

    with scope("p2_qidx"):
        off0 = 0
        for n in PCH:
            pltpu.sync_copy(q_t.at[pl.ds(s * PT + off0, n)],
                            widx.at[pl.ds(0, n)])
            pltpu.sync_copy(q_s.at[pl.ds(s * PT + off0, n)],
                            qwidx.at[pl.ds(0, n)])

            def mk_q(i, _, off0=off0):
                o = i * 16
                t = widx[pl.ds(o, 16)]
                sr = qwidx[pl.ds(o, 16)]
                lin = t * NCN + sr - hbase
                qsidx[pl.ds(off0 + o, 16)] = jnp.where((t >> 11) == c,
                                                       lin, BIG)
                return 0
            lax.fori_loop(0, n // 16, mk_q, 0)
            off0 += n

    cacc = zeros16
    zbase = s * TZ
    for t_sl in range(NSL):
        base = t_sl * SLW

        with scope("p3_zero"):
            def fire_zero(k, _):
                pltpu.async_copy(zbuf,
                                 qmap_sh.at[pl.ds(zbase + k * ZW, ZW)], semz)
                return 0
            lax.fori_loop(0, TZ // ZW, fire_zero, 0)

            def wait_zero(k, _):
                pltpu.make_async_copy(
                    zbuf, qmap_sh.at[pl.ds(zbase + k * ZW, ZW)], semz).wait()
                return 0
            lax.fori_loop(0, TZ // ZW, wait_zero, 0)
        plsc.subcore_barrier()

        with scope("p4_scat"):
            fired = []
            off0 = 0
            for ch, n in enumerate(KCH):
                bk = (ch & 1) * CH
                if len(fired) >= 2:
                    fo, fn, fb_ = fired[len(fired) - 2]
                    pltpu.make_async_copy(
                        ones_b.at[pl.ds(0, fn)],
                        qmap_sh.at[widx.at[pl.ds(fb_, fn)]], sems).wait()

                def mk_w(i, _, off0=off0, bk=bk):
                    o = i * 16
                    d = sidx[pl.ds(off0 + o, 16)] - base
                    ins = (d >= 0) & (d < SLW)
                    pad = WSP + (((off0 + o + lanes) * 32 + w) & 8191)
                    widx[pl.ds(bk + o, 16)] = jnp.where(ins, d, pad)
                    return 0
                lax.fori_loop(0, n // 16, mk_w, 0)
                pltpu.async_copy(ones_b.at[pl.ds(0, n)],
                                 qmap_sh.at[widx.at[pl.ds(bk, n)]],
                                 sems, add=True)
                fired.append((off0, n, bk))
                off0 += n
            for fo, fn, fb_ in fired[len(fired) - 2:]:
                pltpu.make_async_copy(
                    ones_b.at[pl.ds(0, fn)],
                    qmap_sh.at[widx.at[pl.ds(fb_, fn)]], sems).wait()
        plsc.subcore_barrier()

        with scope("p5_gath"):
            live = []
            off0 = 0
            for ch, n in enumerate(PCH):
                bk = (ch & 1) * CH

                def mk_qw(i, _, off0=off0, bk=bk):
                    o = i * 16
                    d = qsidx[pl.ds(off0 + o, 16)] - base
                    ins = (d >= 0) & (d < SLW)
                    pad = RSP + (((off0 + o + lanes) * 32 + s) & 8191)
                    qwidx[pl.ds(bk + o, 16)] = jnp.where(ins, d, pad)
                    return 0
                lax.fori_loop(0, n // 16, mk_qw, 0)
                pltpu.async_copy(qmap_sh.at[qwidx.at[pl.ds(bk, n)]],
                                 qdst.at[pl.ds(bk, n)], semg)
                live.append((n, bk))
                if len(live) == 2:
                    fn, fb_ = live.pop(0)
                    pltpu.make_async_copy(
                        qmap_sh.at[qwidx.at[pl.ds(fb_, fn)]],
                        qdst.at[pl.ds(fb_, fn)], semg).wait()

                    def acc_f(i, a, fb_=fb_):
                        g = qdst[pl.ds(fb_ + i * 16, 16)]
                        return a + jnp.where(g > 0.0, 1.0, 0.0)
                    cacc = lax.fori_loop(0, fn // 16, acc_f, cacc)
                off0 += n
            for fn, fb_ in live:
                pltpu.make_async_copy(
                    qmap_sh.at[qwidx.at[pl.ds(fb_, fn)]],
                    qdst.at[pl.ds(fb_, fn)], semg).wait()

                def acc_f(i, a, fb_=fb_):
                    g = qdst[pl.ds(fb_ + i * 16, 16)]
                    return a + jnp.where(g > 0.0, 1.0, 0.0)
                cacc = lax.fori_loop(0, fn // 16, acc_f, cacc)
        plsc.subcore_barrier()

    cv = [consts_v[pl.ds(j * 16, 16)] for j in range(13)]
    facc = zeros16
    with scope("p6_fine"):
        off0 = 0
        for n in FCH:
            hs = (tx_h, ty_h, tz_h, sx_h, sy_h, sz_h)
            for i, h in enumerate(hs[:4]):
                pltpu.sync_copy(h.at[pl.ds(w * QT + off0, n)],
                                qdst.at[pl.ds(i * 1024, n)])
            for i, h in enumerate(hs[4:]):
                pltpu.sync_copy(h.at[pl.ds(w * QT + off0, n)],
                                ones_b.at[pl.ds(i * 1024, n)])

            def fine(i, fa):
                o = i * 16
                tx = qdst[pl.ds(o, 16)]
                ty = qdst[pl.ds(1024 + o, 16)]
                tz = qdst[pl.ds(2048 + o, 16)]
                sx = qdst[pl.ds(3072 + o, 16)]
                sy = ones_b[pl.ds(o, 16)]
                sz = ones_b[pl.ds(1024 + o, 16)]
                dx = cv[0] * sx + cv[1] * sy + cv[2] * sz + cv[9] - tx
                dy = cv[3] * sx + cv[4] * sy + cv[5] * sz + cv[10] - ty
                dz = cv[6] * sx + cv[7] * sy + cv[8] * sz + cv[11] - tz
                d2 = dx * dx + dy * dy + dz * dz
                return fa + jnp.where(d2 < cv[12], ones16, zeros16)
            facc = lax.fori_loop(0, n // 16, fine, facc)
            off0 += n

    def clr_acc(i, _):
        acc_b[pl.ds(i * 16, 16)] = zeros16
        return 0
    lax.fori_loop(0, 16, clr_acc, 0)
    acc_b[pl.ds(0, 16)] = cacc
    acc_b[pl.ds(128, 16)] = facc
    pltpu.sync_copy(acc_b.at[pl.ds(0, 128)], couts.at[w])
    pltpu.sync_copy(acc_b.at[pl.ds(128, 128)], fouts.at[w])


@jax.jit
def _run(gt_t, gt_s, ovl, q_t, q_s, tx, ty, tz, sx, sy, sz, consts):
    f = pl.kernel(
        _sc_body,
        out_type=(
            jax.ShapeDtypeStruct((32, 128), jnp.float32),
            jax.ShapeDtypeStruct((32, 128), jnp.float32),
        ),
        mesh=_mesh,
        scratch_types=(
            pltpu.VMEM((ZW,), jnp.float32),
            pltpu.VMEM((KT,), jnp.int32),
            pltpu.VMEM((PT,), jnp.int32),
            pltpu.VMEM((2 * CH,), jnp.int32),
            pltpu.VMEM((2 * CH,), jnp.int32),
            pltpu.VMEM((2 * CH,), jnp.float32),
            pltpu.VMEM((CH,), jnp.float32),
            pltpu.VMEM((256,), jnp.float32),
            pltpu.VMEM((208,), jnp.float32),
            pltpu.VMEM_SHARED((TW,), jnp.float32),
            pltpu.SemaphoreType.DMA,
            pltpu.SemaphoreType.DMA,
            pltpu.SemaphoreType.DMA,
        ),
    )
    return f(gt_t, gt_s, ovl, q_t, q_s, tx, ty, tz, sx, sy, sz, consts)


def kernel(tgt_nodes, src_nodes, src_node_feats, gt_node_corr_overlaps,
           gt_node_corr_indices, tgt_node_corr_indices, src_node_corr_indices,
           tgt_corr_points, src_corr_points, rot, trans):
    gti = gt_node_corr_indices.astype(jnp.int32)
    gt_t = jnp.concatenate([gti[:, 0], jnp.zeros((KP - K,), jnp.int32)])
    gt_s = jnp.concatenate([gti[:, 1], jnp.zeros((KP - K,), jnp.int32)])
    ovl = jnp.concatenate([gt_node_corr_overlaps,
                           jnp.zeros((KP - K,), jnp.float32)])
    q_t = jnp.concatenate([tgt_node_corr_indices.astype(jnp.int32),
                           jnp.full((PP - P,), NCN, jnp.int32)])
    q_s = jnp.concatenate([src_node_corr_indices.astype(jnp.int32),
                           jnp.zeros((PP - P,), jnp.int32)])
    tpts = jnp.concatenate([tgt_corr_points,
                            jnp.full((QP - Q, 3), 1e9, jnp.float32)]).T
    spts = jnp.concatenate([src_corr_points,
                            jnp.zeros((QP - Q, 3), jnp.float32)]).T
    consts = (jnp.concatenate([
        rot[0].reshape(9), trans[0].reshape(3),
        jnp.array([0.01], jnp.float32), jnp.zeros((3,), jnp.float32),
    ])[:13].reshape(13, 1) * jnp.ones((1, 16), jnp.float32)).reshape(208)

    couts, fouts = _run(gt_t, gt_s, ovl, q_t, q_s,
                        tpts[0], tpts[1], tpts[2],
                        spts[0], spts[1], spts[2], consts)

    c_precision = jnp.sum(couts) / jnp.float32(P)
    f_precision = jnp.sum(fouts) / jnp.float32(Q)
    fmr = f_precision > 0.05
    num_matches = jnp.array(Q, dtype=jnp.int32)
    return (c_precision, f_precision, fmr, num_matches)


# speedup vs baseline: 4.6452x; 1.1602x over previous
import jax
import jax.numpy as jnp
from jax import lax
from jax.experimental import pallas as pl
from jax.experimental.pallas import tpu as pltpu
from jax.experimental.pallas import tpu_sc as plsc

NCN = 4096
HALFP = NCN * NCN // 2
BIG = 0x40000000

K = 200000
P = 100000
Q = 100000

KT = 13312
KP = KT * 16
PT = 6656
PP = PT * 16
QT = 3200
QP = QT * 32

CH = 2048
KCH = [2048] * 6 + [1024]
PCH = [2048] * 3 + [512]
FCH = [1024] * 3 + [128]

SLW = 1425408
NSL = 6
WSP = SLW
RSP = SLW + 8192
TW = SLW + 16384
ZW = 4096
TZ = TW // 16

_mesh = plsc.VectorSubcoreMesh(
    core_axis_name="c", subcore_axis_name="s", num_cores=2, num_subcores=16)


def _sc_body(gt_t, gt_s, ovl, q_t, q_s, tx_h, ty_h, tz_h, sx_h, sy_h, sz_h,
             consts,
             couts, fouts,
             zbuf, sidx, qsidx, widx, qwidx, qdst, ones_b, acc_b,
             consts_v, qmap_sh, semz, sems, semg):
    c = lax.axis_index("c")
    s = lax.axis_index("s")
    w = c * 16 + s
    lanes = lax.iota(jnp.int32, 16)
    zeros16 = jnp.zeros((16,), jnp.float32)
    ones16 = jnp.ones((16,), jnp.float32)
    scope = jax.named_scope

    pltpu.sync_copy(consts, consts_v)

    def fill_o(i, _):
        ones_b[pl.ds(i * 16, 16)] = ones16
        return 0
    lax.fori_loop(0, CH // 16, fill_o, 0)

    def fill_z(i, _):
        zbuf[pl.ds(i * 16, 16)] = zeros16
        return 0
    lax.fori_loop(0, ZW // 16, fill_z, 0)

    hbase = c * HALFP

    with scope("p1_sidx"):
        off0 = 0
        for n in KCH:
            pltpu.sync_copy(gt_t.at[pl.ds(s * KT + off0, n)],
                            widx.at[pl.ds(0, n)])
            pltpu.sync_copy(gt_s.at[pl.ds(s * KT + off0, n)],
                            qwidx.at[pl.ds(0, n)])
            pltpu.sync_copy(ovl.at[pl.ds(s * KT + off0, n)],
                            qdst.at[pl.ds(0, n)])

            def mk_s(i, _, off0=off0):
                for u in range(4):
                    o = i * 64 + u * 16
                    t = widx[pl.ds(o, 16)]
                    sr = qwidx[pl.ds(o, 16)]
                    ov = qdst[pl.ds(o, 16)]
                    lin = t * NCN + sr - hbase
                    valid = (ov > 0.0) & ((t >> 11) == c)
                    sidx[pl.ds(off0 + o, 16)] = jnp.where(valid, lin, BIG)
                return 0
            lax.fori_loop(0, n // 64, mk_s, 0)
            off0 += n

    with scope("p2_qidx"):
        off0 = 0
        for n in PCH:
            pltpu.sync_copy(q_t.at[pl.ds(s * PT + off0, n)],
                            widx.at[pl.ds(0, n)])
            pltpu.sync_copy(q_s.at[pl.ds(s * PT + off0, n)],
                            qwidx.at[pl.ds(0, n)])

            def mk_q(i, _, off0=off0):
                for u in range(4):
                    o = i * 64 + u * 16
                    t = widx[pl.ds(o, 16)]
                    sr = qwidx[pl.ds(o, 16)]
                    lin = t * NCN + sr - hbase
                    qsidx[pl.ds(off0 + o, 16)] = jnp.where((t >> 11) == c,
                                                           lin, BIG)
                return 0
            lax.fori_loop(0, n // 64, mk_q, 0)
            off0 += n

    cacc = zeros16
    zbase = s * TZ
    for t_sl in range(NSL):
        base = t_sl * SLW

        with scope("p3_zero"):
            def fire_zero(k, _):
                pltpu.async_copy(zbuf,
                                 qmap_sh.at[pl.ds(zbase + k * ZW, ZW)], semz)
                return 0
            lax.fori_loop(0, TZ // ZW, fire_zero, 0)

            def wait_zero(k, _):
                pltpu.make_async_copy(
                    zbuf, qmap_sh.at[pl.ds(zbase + k * ZW, ZW)], semz).wait()
                return 0
            lax.fori_loop(0, TZ // ZW, wait_zero, 0)
        plsc.subcore_barrier()

        with scope("p4_scat"):
            fired = []
            off0 = 0
            for ch, n in enumerate(KCH):
                bk = (ch & 1) * CH
                if len(fired) >= 2:
                    fo, fn, fb_ = fired[len(fired) - 2]
                    pltpu.make_async_copy(
                        ones_b.at[pl.ds(0, fn)],
                        qmap_sh.at[widx.at[pl.ds(fb_, fn)]], sems).wait()

                def mk_w(i, _, off0=off0, bk=bk):
                    for u in range(4):
                        o = i * 64 + u * 16
                        d = sidx[pl.ds(off0 + o, 16)] - base
                        ins = plsc.bitcast(d, jnp.uint32) < jnp.uint32(SLW)
                        pad = (WSP + ((off0 + o + w * 16) & 8176)) + lanes
                        widx[pl.ds(bk + o, 16)] = jnp.where(ins, d, pad)
                    return 0
                lax.fori_loop(0, n // 64, mk_w, 0)
                pltpu.async_copy(ones_b.at[pl.ds(0, n)],
                                 qmap_sh.at[widx.at[pl.ds(bk, n)]],
                                 sems, add=True)
                fired.append((off0, n, bk))
                off0 += n
            for fo, fn, fb_ in fired[len(fired) - 2:]:
                pltpu.make_async_copy(
                    ones_b.at[pl.ds(0, fn)],
                    qmap_sh.at[widx.at[pl.ds(fb_, fn)]], sems).wait()
        plsc.subcore_barrier()

        with scope("p5_gath"):
            live = []
            off0 = 0
            for ch, n in enumerate(PCH):
                bk = (ch & 1) * CH

                def mk_qw(i, _, off0=off0, bk=bk):
                    for u in range(4):
                        o = i * 64 + u * 16
                        d = qsidx[pl.ds(off0 + o, 16)] - base
                        ins = plsc.bitcast(d, jnp.uint32) < jnp.uint32(SLW)
                        pad = (RSP + ((off0 + o + s * 16) & 8176)) + lanes
                        qwidx[pl.ds(bk + o, 16)] = jnp.where(ins, d, pad)
                    return 0
                lax.fori_loop(0, n // 64, mk_qw, 0)
                pltpu.async_copy(qmap_sh.at[qwidx.at[pl.ds(bk, n)]],
                                 qdst.at[pl.ds(bk, n)], semg)
                live.append((n, bk))
                if len(live) == 2:
                    fn, fb_ = live.pop(0)
                    pltpu.make_async_copy(
                        qmap_sh.at[qwidx.at[pl.ds(fb_, fn)]],
                        qdst.at[pl.ds(fb_, fn)], semg).wait()

                    def acc_f(i, a, fb_=fb_):
                        for u in range(4):
                            g = qdst[pl.ds(fb_ + i * 64 + u * 16, 16)]
                            a = a + jnp.where(g > 0.0, 1.0, 0.0)
                        return a
                    cacc = lax.fori_loop(0, fn // 64, acc_f, cacc)
                off0 += n
            for fn, fb_ in live:
                pltpu.make_async_copy(
                    qmap_sh.at[qwidx.at[pl.ds(fb_, fn)]],
                    qdst.at[pl.ds(fb_, fn)], semg).wait()

                def acc_f(i, a, fb_=fb_):
                    for u in range(4):
                        g = qdst[pl.ds(fb_ + i * 64 + u * 16, 16)]
                        a = a + jnp.where(g > 0.0, 1.0, 0.0)
                    return a
                cacc = lax.fori_loop(0, fn // 64, acc_f, cacc)
        plsc.subcore_barrier()

    cv = [consts_v[pl.ds(j * 16, 16)] for j in range(13)]
    facc = zeros16
    with scope("p6_fine"):
        off0 = 0
        for n in FCH:
            hs = (tx_h, ty_h, tz_h, sx_h, sy_h, sz_h)
            for i, h in enumerate(hs[:4]):
                pltpu.sync_copy(h.at[pl.ds(w * QT + off0, n)],
                                qdst.at[pl.ds(i * 1024, n)])
            for i, h in enumerate(hs[4:]):
                pltpu.sync_copy(h.at[pl.ds(w * QT + off0, n)],
                                ones_b.at[pl.ds(i * 1024, n)])

            def fine(i, fa):
                for u in range(4):
                    o = i * 64 + u * 16
                    tx = qdst[pl.ds(o, 16)]
                    ty = qdst[pl.ds(1024 + o, 16)]
                    tz = qdst[pl.ds(2048 + o, 16)]
                    sx = qdst[pl.ds(3072 + o, 16)]
                    sy = ones_b[pl.ds(o, 16)]
                    sz = ones_b[pl.ds(1024 + o, 16)]
                    dx = cv[0] * sx + cv[1] * sy + cv[2] * sz + cv[9] - tx
                    dy = cv[3] * sx + cv[4] * sy + cv[5] * sz + cv[10] - ty
                    dz = cv[6] * sx + cv[7] * sy + cv[8] * sz + cv[11] - tz
                    d2 = dx * dx + dy * dy + dz * dz
                    fa = fa + jnp.where(d2 < cv[12], ones16, zeros16)
                return fa
            facc = lax.fori_loop(0, n // 64, fine, facc)
            off0 += n

    def clr_acc(i, _):
        acc_b[pl.ds(i * 16, 16)] = zeros16
        return 0
    lax.fori_loop(0, 16, clr_acc, 0)
    acc_b[pl.ds(0, 16)] = cacc
    acc_b[pl.ds(128, 16)] = facc
    pltpu.sync_copy(acc_b.at[pl.ds(0, 128)], couts.at[w])
    pltpu.sync_copy(acc_b.at[pl.ds(128, 128)], fouts.at[w])


@jax.jit
def _run(gt_t, gt_s, ovl, q_t, q_s, tx, ty, tz, sx, sy, sz, consts):
    f = pl.kernel(
        _sc_body,
        out_type=(
            jax.ShapeDtypeStruct((32, 128), jnp.float32),
            jax.ShapeDtypeStruct((32, 128), jnp.float32),
        ),
        mesh=_mesh,
        scratch_types=(
            pltpu.VMEM((ZW,), jnp.float32),
            pltpu.VMEM((KT,), jnp.int32),
            pltpu.VMEM((PT,), jnp.int32),
            pltpu.VMEM((2 * CH,), jnp.int32),
            pltpu.VMEM((2 * CH,), jnp.int32),
            pltpu.VMEM((2 * CH,), jnp.float32),
            pltpu.VMEM((CH,), jnp.float32),
            pltpu.VMEM((256,), jnp.float32),
            pltpu.VMEM((208,), jnp.float32),
            pltpu.VMEM_SHARED((TW,), jnp.float32),
            pltpu.SemaphoreType.DMA,
            pltpu.SemaphoreType.DMA,
            pltpu.SemaphoreType.DMA,
        ),
    )
    return f(gt_t, gt_s, ovl, q_t, q_s, tx, ty, tz, sx, sy, sz, consts)


def kernel(tgt_nodes, src_nodes, src_node_feats, gt_node_corr_overlaps,
           gt_node_corr_indices, tgt_node_corr_indices, src_node_corr_indices,
           tgt_corr_points, src_corr_points, rot, trans):
    gti = gt_node_corr_indices.astype(jnp.int32)
    gt_t = jnp.concatenate([gti[:, 0], jnp.zeros((KP - K,), jnp.int32)])
    gt_s = jnp.concatenate([gti[:, 1], jnp.zeros((KP - K,), jnp.int32)])
    ovl = jnp.concatenate([gt_node_corr_overlaps,
                           jnp.zeros((KP - K,), jnp.float32)])
    q_t = jnp.concatenate([tgt_node_corr_indices.astype(jnp.int32),
                           jnp.full((PP - P,), NCN, jnp.int32)])
    q_s = jnp.concatenate([src_node_corr_indices.astype(jnp.int32),
                           jnp.zeros((PP - P,), jnp.int32)])
    tpts = jnp.concatenate([tgt_corr_points,
                            jnp.full((QP - Q, 3), 1e9, jnp.float32)]).T
    spts = jnp.concatenate([src_corr_points,
                            jnp.zeros((QP - Q, 3), jnp.float32)]).T
    consts = (jnp.concatenate([
        rot[0].reshape(9), trans[0].reshape(3),
        jnp.array([0.01], jnp.float32), jnp.zeros((3,), jnp.float32),
    ])[:13].reshape(13, 1) * jnp.ones((1, 16), jnp.float32)).reshape(208)

    couts, fouts = _run(gt_t, gt_s, ovl, q_t, q_s,
                        tpts[0], tpts[1], tpts[2],
                        spts[0], spts[1], spts[2], consts)

    c_precision = jnp.sum(couts) / jnp.float32(P)
    f_precision = jnp.sum(fouts) / jnp.float32(Q)
    fmr = f_precision > 0.05
    num_matches = jnp.array(Q, dtype=jnp.int32)
    return (c_precision, f_precision, fmr, num_matches)


# speedup vs baseline: 5.1334x; 1.1051x over previous
import jax
import jax.numpy as jnp
from jax import lax
from jax.experimental import pallas as pl
from jax.experimental.pallas import tpu as pltpu
from jax.experimental.pallas import tpu_sc as plsc

NCN = 4096
HALFP = NCN * NCN // 2
BIG = 0x40000000

K = 200000
P = 100000
Q = 100000

KT = 13312
KP = KT * 16
PT = 6656
PP = PT * 16
QT = 3200
QP = QT * 32

CH = 2048
KCH = [2048] * 6 + [1024]
PCH = [2048] * 3 + [512]
FCH = [1024] * 3 + [128]

SLW = 1425408
NSL = 6
WSP = SLW
RSP = SLW + 8192
TW = SLW + 16384
ZW = 4096
TZ = TW // 16

_mesh = plsc.VectorSubcoreMesh(
    core_axis_name="c", subcore_axis_name="s", num_cores=2, num_subcores=16)


def _sc_body(gt_t, gt_s, ovl, q_t, q_s, tx_h, ty_h, tz_h, sx_h, sy_h, sz_h,
             consts,
             couts, fouts,
             zbuf, sidx, qsidx, widx, qwidx, qdst, ones_b, acc_b,
             consts_v, qmap_sh, semz, sems, semg):
    c = lax.axis_index("c")
    s = lax.axis_index("s")
    w = c * 16 + s
    lanes = lax.iota(jnp.int32, 16)
    zeros16 = jnp.zeros((16,), jnp.float32)
    ones16 = jnp.ones((16,), jnp.float32)
    scope = jax.named_scope

    pltpu.sync_copy(consts, consts_v)

    def fill_o(i, _):
        ones_b[pl.ds(i * 16, 16)] = ones16
        return 0
    lax.fori_loop(0, CH // 16, fill_o, 0)

    def fill_z(i, _):
        zbuf[pl.ds(i * 16, 16)] = zeros16
        return 0
    lax.fori_loop(0, ZW // 16, fill_z, 0)

    hbase = c * HALFP

    with scope("p1_sidx"):
        koff = [sum(KCH[:i]) for i in range(len(KCH))]

        def k_fire(ch):
            n, off, bk = KCH[ch], koff[ch], (ch & 1) * CH
            sm = sems if ch & 1 else semz
            pltpu.async_copy(gt_t.at[pl.ds(s * KT + off, n)],
                             widx.at[pl.ds(bk, n)], sm)
            pltpu.async_copy(gt_s.at[pl.ds(s * KT + off, n)],
                             qwidx.at[pl.ds(bk, n)], sm)
            pltpu.async_copy(ovl.at[pl.ds(s * KT + off, n)],
                             qdst.at[pl.ds(bk, n)], sm)

        def k_wait(ch):
            n, off, bk = KCH[ch], koff[ch], (ch & 1) * CH
            sm = sems if ch & 1 else semz
            pltpu.make_async_copy(gt_t.at[pl.ds(s * KT + off, n)],
                                  widx.at[pl.ds(bk, n)], sm).wait()
            pltpu.make_async_copy(gt_s.at[pl.ds(s * KT + off, n)],
                                  qwidx.at[pl.ds(bk, n)], sm).wait()
            pltpu.make_async_copy(ovl.at[pl.ds(s * KT + off, n)],
                                  qdst.at[pl.ds(bk, n)], sm).wait()

        k_fire(0)
        for ch, n in enumerate(KCH):
            if ch + 1 < len(KCH):
                k_fire(ch + 1)
            k_wait(ch)
            bk = (ch & 1) * CH

            def mk_s(i, _, off0=koff[ch], bk=bk):
                for u in range(4):
                    o = i * 64 + u * 16
                    t = widx[pl.ds(bk + o, 16)]
                    sr = qwidx[pl.ds(bk + o, 16)]
                    ov = qdst[pl.ds(bk + o, 16)]
                    lin = t * NCN + sr - hbase
                    valid = (ov > 0.0) & ((t >> 11) == c)
                    sidx[pl.ds(off0 + o, 16)] = jnp.where(valid, lin, BIG)
                return 0
            lax.fori_loop(0, n // 64, mk_s, 0)

    with scope("p2_qidx"):
        poff = [sum(PCH[:i]) for i in range(len(PCH))]

        def q_fire(ch):
            n, off, bk = PCH[ch], poff[ch], (ch & 1) * CH
            sm = sems if ch & 1 else semz
            pltpu.async_copy(q_t.at[pl.ds(s * PT + off, n)],
                             widx.at[pl.ds(bk, n)], sm)
            pltpu.async_copy(q_s.at[pl.ds(s * PT + off, n)],
                             qwidx.at[pl.ds(bk, n)], sm)

        def q_wait(ch):
            n, off, bk = PCH[ch], poff[ch], (ch & 1) * CH
            sm = sems if ch & 1 else semz
            pltpu.make_async_copy(q_t.at[pl.ds(s * PT + off, n)],
                                  widx.at[pl.ds(bk, n)], sm).wait()
            pltpu.make_async_copy(q_s.at[pl.ds(s * PT + off, n)],
                                  qwidx.at[pl.ds(bk, n)], sm).wait()

        q_fire(0)
        for ch, n in enumerate(PCH):
            if ch + 1 < len(PCH):
                q_fire(ch + 1)
            q_wait(ch)
            bk = (ch & 1) * CH

            def mk_q(i, _, off0=poff[ch], bk=bk):
                for u in range(4):
                    o = i * 64 + u * 16
                    t = widx[pl.ds(bk + o, 16)]
                    sr = qwidx[pl.ds(bk + o, 16)]
                    lin = t * NCN + sr - hbase
                    qsidx[pl.ds(off0 + o, 16)] = jnp.where((t >> 11) == c,
                                                           lin, BIG)
                return 0
            lax.fori_loop(0, n // 64, mk_q, 0)

    cacc = zeros16
    zbase = s * TZ
    for t_sl in range(NSL):
        base = t_sl * SLW

        with scope("p3_zero"):
            def fire_zero(k, _):
                pltpu.async_copy(zbuf,
                                 qmap_sh.at[pl.ds(zbase + k * ZW, ZW)], semz)
                return 0
            lax.fori_loop(0, TZ // ZW, fire_zero, 0)

            def wait_zero(k, _):
                pltpu.make_async_copy(
                    zbuf, qmap_sh.at[pl.ds(zbase + k * ZW, ZW)], semz).wait()
                return 0
            lax.fori_loop(0, TZ // ZW, wait_zero, 0)
        plsc.subcore_barrier()

        with scope("p4_scat"):
            fired = []
            off0 = 0
            for ch, n in enumerate(KCH):
                bk = (ch & 1) * CH
                if len(fired) >= 2:
                    fo, fn, fb_ = fired[len(fired) - 2]
                    pltpu.make_async_copy(
                        ones_b.at[pl.ds(0, fn)],
                        qmap_sh.at[widx.at[pl.ds(fb_, fn)]], sems).wait()

                def mk_w(i, _, off0=off0, bk=bk):
                    for u in range(4):
                        o = i * 64 + u * 16
                        d = sidx[pl.ds(off0 + o, 16)] - base
                        ins = plsc.bitcast(d, jnp.uint32) < jnp.uint32(SLW)
                        pad = (WSP + ((off0 + o + w * 16) & 8176)) + lanes
                        widx[pl.ds(bk + o, 16)] = jnp.where(ins, d, pad)
                    return 0
                lax.fori_loop(0, n // 64, mk_w, 0)
                pltpu.async_copy(ones_b.at[pl.ds(0, n)],
                                 qmap_sh.at[widx.at[pl.ds(bk, n)]],
                                 sems, add=True)
                fired.append((off0, n, bk))
                off0 += n
            for fo, fn, fb_ in fired[len(fired) - 2:]:
                pltpu.make_async_copy(
                    ones_b.at[pl.ds(0, fn)],
                    qmap_sh.at[widx.at[pl.ds(fb_, fn)]], sems).wait()
        plsc.subcore_barrier()

        with scope("p5_gath"):
            live = []
            off0 = 0
            for ch, n in enumerate(PCH):
                bk = (ch & 1) * CH

                def mk_qw(i, _, off0=off0, bk=bk):
                    for u in range(4):
                        o = i * 64 + u * 16
                        d = qsidx[pl.ds(off0 + o, 16)] - base
                        ins = plsc.bitcast(d, jnp.uint32) < jnp.uint32(SLW)
                        pad = (RSP + ((off0 + o + s * 16) & 8176)) + lanes
                        qwidx[pl.ds(bk + o, 16)] = jnp.where(ins, d, pad)
                    return 0
                lax.fori_loop(0, n // 64, mk_qw, 0)
                pltpu.async_copy(qmap_sh.at[qwidx.at[pl.ds(bk, n)]],
                                 qdst.at[pl.ds(bk, n)], semg)
                live.append((n, bk))
                if len(live) == 2:
                    fn, fb_ = live.pop(0)
                    pltpu.make_async_copy(
                        qmap_sh.at[qwidx.at[pl.ds(fb_, fn)]],
                        qdst.at[pl.ds(fb_, fn)], semg).wait()

                    def acc_f(i, a, fb_=fb_):
                        for u in range(4):
                            g = qdst[pl.ds(fb_ + i * 64 + u * 16, 16)]
                            a = a + jnp.where(g > 0.0, 1.0, 0.0)
                        return a
                    cacc = lax.fori_loop(0, fn // 64, acc_f, cacc)
                off0 += n
            for fn, fb_ in live:
                pltpu.make_async_copy(
                    qmap_sh.at[qwidx.at[pl.ds(fb_, fn)]],
                    qdst.at[pl.ds(fb_, fn)], semg).wait()

                def acc_f(i, a, fb_=fb_):
                    for u in range(4):
                        g = qdst[pl.ds(fb_ + i * 64 + u * 16, 16)]
                        a = a + jnp.where(g > 0.0, 1.0, 0.0)
                    return a
                cacc = lax.fori_loop(0, fn // 64, acc_f, cacc)
        plsc.subcore_barrier()

    cv = [consts_v[pl.ds(j * 16, 16)] for j in range(13)]
    facc = zeros16
    with scope("p6_fine"):
        off0 = 0
        for n in FCH:
            hs = (tx_h, ty_h, tz_h, sx_h, sy_h, sz_h)
            for i, h in enumerate(hs[:4]):
                pltpu.sync_copy(h.at[pl.ds(w * QT + off0, n)],
                                qdst.at[pl.ds(i * 1024, n)])
            for i, h in enumerate(hs[4:]):
                pltpu.sync_copy(h.at[pl.ds(w * QT + off0, n)],
                                ones_b.at[pl.ds(i * 1024, n)])

            def fine(i, fa):
                for u in range(4):
                    o = i * 64 + u * 16
                    tx = qdst[pl.ds(o, 16)]
                    ty = qdst[pl.ds(1024 + o, 16)]
                    tz = qdst[pl.ds(2048 + o, 16)]
                    sx = qdst[pl.ds(3072 + o, 16)]
                    sy = ones_b[pl.ds(o, 16)]
                    sz = ones_b[pl.ds(1024 + o, 16)]
                    dx = cv[0] * sx + cv[1] * sy + cv[2] * sz + cv[9] - tx
                    dy = cv[3] * sx + cv[4] * sy + cv[5] * sz + cv[10] - ty
                    dz = cv[6] * sx + cv[7] * sy + cv[8] * sz + cv[11] - tz
                    d2 = dx * dx + dy * dy + dz * dz
                    fa = fa + jnp.where(d2 < cv[12], ones16, zeros16)
                return fa
            facc = lax.fori_loop(0, n // 64, fine, facc)
            off0 += n

    def clr_acc(i, _):
        acc_b[pl.ds(i * 16, 16)] = zeros16
        return 0
    lax.fori_loop(0, 16, clr_acc, 0)
    acc_b[pl.ds(0, 16)] = cacc
    acc_b[pl.ds(128, 16)] = facc
    pltpu.sync_copy(acc_b.at[pl.ds(0, 128)], couts.at[w])
    pltpu.sync_copy(acc_b.at[pl.ds(128, 128)], fouts.at[w])


@jax.jit
def _run(gt_t, gt_s, ovl, q_t, q_s, tx, ty, tz, sx, sy, sz, consts):
    f = pl.kernel(
        _sc_body,
        out_type=(
            jax.ShapeDtypeStruct((32, 128), jnp.float32),
            jax.ShapeDtypeStruct((32, 128), jnp.float32),
        ),
        mesh=_mesh,
        scratch_types=(
            pltpu.VMEM((ZW,), jnp.float32),
            pltpu.VMEM((KT,), jnp.int32),
            pltpu.VMEM((PT,), jnp.int32),
            pltpu.VMEM((2 * CH,), jnp.int32),
            pltpu.VMEM((2 * CH,), jnp.int32),
            pltpu.VMEM((2 * CH,), jnp.float32),
            pltpu.VMEM((CH,), jnp.float32),
            pltpu.VMEM((256,), jnp.float32),
            pltpu.VMEM((208,), jnp.float32),
            pltpu.VMEM_SHARED((TW,), jnp.float32),
            pltpu.SemaphoreType.DMA,
            pltpu.SemaphoreType.DMA,
            pltpu.SemaphoreType.DMA,
        ),
    )
    return f(gt_t, gt_s, ovl, q_t, q_s, tx, ty, tz, sx, sy, sz, consts)


def kernel(tgt_nodes, src_nodes, src_node_feats, gt_node_corr_overlaps,
           gt_node_corr_indices, tgt_node_corr_indices, src_node_corr_indices,
           tgt_corr_points, src_corr_points, rot, trans):
    gti = gt_node_corr_indices.astype(jnp.int32)
    gt_t = jnp.concatenate([gti[:, 0], jnp.zeros((KP - K,), jnp.int32)])
    gt_s = jnp.concatenate([gti[:, 1], jnp.zeros((KP - K,), jnp.int32)])
    ovl = jnp.concatenate([gt_node_corr_overlaps,
                           jnp.zeros((KP - K,), jnp.float32)])
    q_t = jnp.concatenate([tgt_node_corr_indices.astype(jnp.int32),
                           jnp.full((PP - P,), NCN, jnp.int32)])
    q_s = jnp.concatenate([src_node_corr_indices.astype(jnp.int32),
                           jnp.zeros((PP - P,), jnp.int32)])
    tpts = jnp.concatenate([tgt_corr_points,
                            jnp.full((QP - Q, 3), 1e9, jnp.float32)]).T
    spts = jnp.concatenate([src_corr_points,
                            jnp.zeros((QP - Q, 3), jnp.float32)]).T
    consts = (jnp.concatenate([
        rot[0].reshape(9), trans[0].reshape(3),
        jnp.array([0.01], jnp.float32), jnp.zeros((3,), jnp.float32),
    ])[:13].reshape(13, 1) * jnp.ones((1, 16), jnp.float32)).reshape(208)

    couts, fouts = _run(gt_t, gt_s, ovl, q_t, q_s,
                        tpts[0], tpts[1], tpts[2],
                        spts[0], spts[1], spts[2], consts)

    c_precision = jnp.sum(couts) / jnp.float32(P)
    f_precision = jnp.sum(fouts) / jnp.float32(Q)
    fmr = f_precision > 0.05
    num_matches = jnp.array(Q, dtype=jnp.int32)
    return (c_precision, f_precision, fmr, num_matches)
